# baseline (speedup 1.0000x reference)
import functools

import jax
import jax.numpy as jnp
from jax import lax
from jax.experimental import pallas as pl
from jax.experimental.pallas import tpu as pltpu
from jax.experimental.pallas import tpu_sc as plsc

_DECAY = 0.99
_CH = 128
_SCP = pltpu.CompilerParams(use_tc_tiling_on_sc=False)


def _wid():
    return lax.axis_index("s") * 2 + lax.axis_index("c")


def _mesh():
    return plsc.VectorSubcoreMesh(core_axis_name="c", subcore_axis_name="s")


def _modulate(val, W):
    B, D = val.shape
    BLK = 2048

    def body(val_ref, w_ref, o_ref):
        o_ref[...] = jnp.tanh(
            jnp.dot(val_ref[...], w_ref[...], preferred_element_type=jnp.float32)
        )

    return pl.pallas_call(
        body,
        grid=(B // BLK,),
        in_specs=[
            pl.BlockSpec((BLK, D), lambda i: (i, 0)),
            pl.BlockSpec((D, D), lambda i: (0, 0)),
        ],
        out_specs=pl.BlockSpec((BLK, D), lambda i: (i, 0)),
        out_shape=jax.ShapeDtypeStruct((B, D), jnp.float32),
    )(val, W)


def _make_winner_scatter(V, B, NW, n_ch):
    @functools.partial(
        pl.kernel,
        mesh=_mesh(),
        compiler_params=_SCP,
        out_type=jax.ShapeDtypeStruct((V, 16), jnp.int32),
        scratch_types=[
            pltpu.VMEM((n_ch, _CH), jnp.int32),
            pltpu.VMEM((n_ch * _CH, 16), jnp.int32),
            pltpu.SemaphoreType.DMA,
        ],
    )
    def k(idx_hbm, rid_hbm, s2_hbm, idx_v, rid_v, sem):
        wid = _wid()
        pltpu.sync_copy(idx_hbm.at[pl.ds(wid * n_ch, n_ch)], idx_v)
        pltpu.sync_copy(rid_hbm.at[pl.ds(wid * n_ch * _CH, n_ch * _CH)], rid_v)
        cps = [
            pltpu.async_copy(
                rid_v.at[pl.ds(c * _CH, _CH)], s2_hbm.at[idx_v.at[c]], sem
            )
            for c in range(n_ch)
        ]
        for cp in cps:
            cp.wait()

    return k


def _make_win_gather(V, B, NW, n_ch):
    b_per_w = n_ch * _CH

    @functools.partial(
        pl.kernel,
        mesh=_mesh(),
        compiler_params=_SCP,
        out_type=jax.ShapeDtypeStruct((B, 16), jnp.int32),
        scratch_types=[
            pltpu.VMEM((n_ch, _CH), jnp.int32),
            pltpu.VMEM((b_per_w, 16), jnp.int32),
            pltpu.SemaphoreType.DMA,
        ],
    )
    def k(s2_hbm, idx_hbm, w16_hbm, idx_v, w16_v, sem):
        wid = _wid()
        pltpu.sync_copy(idx_hbm.at[pl.ds(wid * n_ch, n_ch)], idx_v)
        cps = [
            pltpu.async_copy(
                s2_hbm.at[idx_v.at[c]], w16_v.at[pl.ds(c * _CH, _CH)], sem
            )
            for c in range(n_ch)
        ]
        for cp in cps:
            cp.wait()
        pltpu.sync_copy(w16_v, w16_hbm.at[pl.ds(wid * b_per_w, b_per_w)])

    return k


def _make_accumulate(V, B, D, NW, n_ch):
    b_per_w = n_ch * _CH
    rows_per_tile = B // 16

    @functools.partial(
        pl.kernel,
        mesh=_mesh(),
        compiler_params=_SCP,
        out_type=[
            jax.ShapeDtypeStruct((B, D), jnp.float32),
            jax.ShapeDtypeStruct((B, D), jnp.float32),
        ],
        scratch_types=[
            pltpu.VMEM((n_ch, _CH), jnp.int32),
            pltpu.VMEM((b_per_w, D), jnp.float32),
            pltpu.VMEM_SHARED((B, D), jnp.float32),
            pltpu.SemaphoreType.DMA,
        ],
    )
    def k(win_hbm, w_hbm, z_hbm, accA, accB, win_v, w_v, acc_sp, sem):
        cc = lax.axis_index("c")
        ss = lax.axis_index("s")
        wid = ss * 2 + cc
        pltpu.sync_copy(
            z_hbm.at[pl.ds(ss * rows_per_tile, rows_per_tile)],
            acc_sp.at[pl.ds(ss * rows_per_tile, rows_per_tile)],
        )
        pltpu.sync_copy(win_hbm.at[wid], win_v)
        pltpu.sync_copy(w_hbm.at[pl.ds(wid * b_per_w, b_per_w)], w_v)
        plsc.subcore_barrier()
        for c in range(n_ch):
            pltpu.sync_copy(
                w_v.at[pl.ds(c * _CH, _CH)], acc_sp.at[win_v.at[c]], add=True
            )
        plsc.subcore_barrier()

        @pl.when(cc == 0)
        def _():
            pltpu.sync_copy(
                acc_sp.at[pl.ds(ss * rows_per_tile, rows_per_tile)],
                accA.at[pl.ds(ss * rows_per_tile, rows_per_tile)],
            )

        @pl.when(cc == 1)
        def _():
            pltpu.sync_copy(
                acc_sp.at[pl.ds(ss * rows_per_tile, rows_per_tile)],
                accB.at[pl.ds(ss * rows_per_tile, rows_per_tile)],
            )

    return k


def _tc_detile(memT, F0):
    D, V = memT.shape
    VA = (V // 128) * 128
    NCB = 12
    BLKC = VA // NCB
    FH = D - F0

    def body(memT_ref, out_ref, buf0, buf1, buf2, buf3, s0, s1, s2, s3):
        cb = pl.program_id(0)
        bufs = (buf0, buf1, buf2, buf3)
        sems = (s0, s1, s2, s3)
        rs = []
        for fg in range(FH // 8):
            rs.append(
                pltpu.make_async_copy(
                    memT_ref.at[pl.ds(F0 + fg * 8, 8), pl.ds(cb * BLKC, BLKC)],
                    bufs[fg],
                    sems[fg],
                )
            )
            rs[fg].start()
        ws = []
        for fg in range(FH // 8):
            rs[fg].wait()
            for fi in range(8):
                f = fg * 8 + fi
                w = pltpu.make_async_copy(
                    bufs[fg].at[fi],
                    out_ref.at[pl.ds(f * VA + cb * BLKC, BLKC)],
                    sems[fg],
                )
                w.start()
                ws.append(w)
        for w in ws:
            w.wait()

    return pl.pallas_call(
        body,
        grid=(NCB,),
        in_specs=[pl.BlockSpec(memory_space=pl.ANY)],
        out_specs=pl.BlockSpec(memory_space=pl.ANY),
        out_shape=jax.ShapeDtypeStruct((FH * VA,), jnp.float32),
        scratch_shapes=[
            pltpu.VMEM((8, BLKC), jnp.float32),
            pltpu.VMEM((8, BLKC), jnp.float32),
            pltpu.VMEM((8, BLKC), jnp.float32),
            pltpu.VMEM((8, BLKC), jnp.float32),
            pltpu.SemaphoreType.DMA,
            pltpu.SemaphoreType.DMA,
            pltpu.SemaphoreType.DMA,
            pltpu.SemaphoreType.DMA,
        ],
    )(memT)


def _make_detile(V, D, NW, FH):
    NG = (V // 128) // NW
    EX = (V // 128) % NW
    W_BIG = (NG + 1) * 128
    W_SML = NG * 128
    TAIL = V - (V // 128) * 128

    @functools.partial(
        pl.kernel,
        mesh=_mesh(),
        compiler_params=pltpu.CompilerParams(use_tc_tiling_on_sc=True),
        out_type=jax.ShapeDtypeStruct((FH * V,), jnp.float32),
        scratch_types=[
            pltpu.VMEM((W_BIG,), jnp.float32),
            pltpu.VMEM((W_BIG,), jnp.float32),
            pltpu.VMEM((W_BIG,), jnp.float32),
            pltpu.VMEM((W_BIG,), jnp.float32),
            pltpu.SemaphoreType.DMA,
            pltpu.SemaphoreType.DMA,
            pltpu.SemaphoreType.DMA,
            pltpu.SemaphoreType.DMA,
        ],
    )
    def k(memT_hbm, memf_hbm, buf0, buf1, buf2, buf3, sem0, sem1, sem2, sem3):
        wid = _wid()
        base_big = wid * W_BIG
        base_sml = EX * W_BIG + (wid - EX) * W_SML

        def do(W, base):
            bufs = (buf0, buf1, buf2, buf3)
            sems = (sem0, sem1, sem2, sem3)

            def step(f4, carry):
                rs = []
                for i in range(4):
                    rs.append(pltpu.async_copy(
                        memT_hbm.at[4 * f4 + i].at[pl.ds(base, W)],
                        bufs[i].at[pl.ds(0, W)], sems[i]))
                ws = []
                for i in range(4):
                    rs[i].wait()
                    ws.append(pltpu.async_copy(
                        bufs[i].at[pl.ds(0, W)],
                        memf_hbm.at[pl.ds((4 * f4 + i) * V + base, W)],
                        sems[i]))
                for w in ws:
                    w.wait()
                return carry

            lax.fori_loop(0, FH // 4, step, 0)

        @pl.when(wid < EX)
        def _():
            do(W_BIG, base_big)

        @pl.when(wid >= EX)
        def _():
            do(W_SML, base_sml)

    return k


def _make_gather(V, D, B, NW, n_ch, F0):
    b_per_w = n_ch * _CH
    VA = (V // 128) * 128
    F_UNROLL = 8

    @functools.partial(
        pl.kernel,
        mesh=_mesh(),
        compiler_params=_SCP,
        out_type=[
            jax.ShapeDtypeStruct((D, B), jnp.float32),
            jax.ShapeDtypeStruct((B, D), jnp.float32),
            jax.ShapeDtypeStruct((B, D), jnp.float32),
        ],
        scratch_types=[
            pltpu.VMEM((n_ch, _CH), jnp.int32),
            pltpu.VMEM((n_ch, _CH), jnp.int32),
            pltpu.VMEM((F_UNROLL * n_ch, _CH), jnp.int32),
            pltpu.VMEM((D, b_per_w), jnp.float32),
            pltpu.VMEM((b_per_w, D), jnp.float32),
            pltpu.VMEM((b_per_w, D), jnp.float32),
            pltpu.SemaphoreType.DMA,
            pltpu.SemaphoreType.DMA,
        ],
    )
    def k(memf_hbm, memfB_hbm, idx_hbm, win_hbm, accA, accB,
          gT_hbm, a_hbm, b_hbm,
          idx_v, win_v, idxo_v, g_v, a_v, b_v, sem, sem2):
        wid = _wid()
        pltpu.sync_copy(idx_hbm.at[pl.ds(wid * n_ch, n_ch)], idx_v)
        pltpu.sync_copy(win_hbm.at[wid], win_v)
        acps = []
        for c in range(n_ch):
            acps.append(
                pltpu.async_copy(
                    accA.at[win_v.at[c]], a_v.at[pl.ds(c * _CH, _CH)], sem2
                )
            )
            acps.append(
                pltpu.async_copy(
                    accB.at[win_v.at[c]], b_v.at[pl.ds(c * _CH, _CH)], sem2
                )
            )

        def make_fblock(lo_half):
            def fblock(fb, carry):
                cps = []
                for ff in range(F_UNROLL):
                    for c in range(n_ch):
                        for q in range(_CH // 16):
                            sl = pl.ds(q * 16, 16)
                            if lo_half:
                                f = fb * F_UNROLL + ff
                                idxo_v[ff * n_ch + c, sl] = (
                                    idx_v[c, sl] + f * V
                                )
                            else:
                                f = F0 + fb * F_UNROLL + ff
                                idxo_v[ff * n_ch + c, sl] = (
                                    jnp.minimum(idx_v[c, sl], VA - 1)
                                    + (f - F0) * VA
                                )
                    for c in range(n_ch):
                        src = memf_hbm if lo_half else memfB_hbm
                        fo = (fb * F_UNROLL + ff) if lo_half \
                            else (F0 + fb * F_UNROLL + ff)
                        cps.append(
                            pltpu.async_copy(
                                src.at[idxo_v.at[ff * n_ch + c]],
                                g_v.at[fo].at[pl.ds(c * _CH, _CH)],
                                sem,
                            )
                        )
                for cp in cps:
                    cp.wait()
                return carry

            return fblock

        lax.fori_loop(0, F0 // F_UNROLL, make_fblock(True), 0)
        lax.fori_loop(0, (D - F0) // F_UNROLL, make_fblock(False), 0)
        pltpu.sync_copy(g_v, gT_hbm.at[:, pl.ds(wid * b_per_w, b_per_w)])
        for cp in acps:
            cp.wait()
        pltpu.sync_copy(a_v, a_hbm.at[pl.ds(wid * b_per_w, b_per_w)])
        pltpu.sync_copy(b_v, b_hbm.at[pl.ds(wid * b_per_w, b_per_w)])

    return k


def _final_combine(gT, a, b, idx2, tail, V):
    D, B = gT.shape
    T = tail.shape[0]
    BLK = 2048
    eye = jnp.eye(D, dtype=jnp.float32)

    def body(gT_ref, a_ref, b_ref, eye_ref, idx_ref, tail_ref, o_ref):
        gt = lax.dot_general(
            gT_ref[...], eye_ref[...],
            (((0,), (0,)), ((), ())),
            precision=lax.Precision.HIGHEST,
            preferred_element_type=jnp.float32,
        )
        rel = idx_ref[...] - (V - T)
        oneh = (rel == lax.broadcasted_iota(jnp.int32, (BLK, T), 1)
                ).astype(jnp.float32)
        tg = lax.dot_general(
            oneh, tail_ref[...],
            (((1,), (0,)), ((), ())),
            precision=lax.Precision.HIGHEST,
            preferred_element_type=jnp.float32,
        )
        mask = rel >= 0
        o_ref[...] = _DECAY * jnp.where(mask, tg, gt) + a_ref[...] + b_ref[...]

    return pl.pallas_call(
        body,
        grid=(B // BLK,),
        in_specs=[
            pl.BlockSpec((D, BLK), lambda i: (0, i)),
            pl.BlockSpec((BLK, D), lambda i: (i, 0)),
            pl.BlockSpec((BLK, D), lambda i: (i, 0)),
            pl.BlockSpec((D, D), lambda i: (0, 0)),
            pl.BlockSpec((BLK, 1), lambda i: (i, 0)),
            pl.BlockSpec((T, D), lambda i: (0, 0)),
        ],
        out_specs=pl.BlockSpec((BLK, D), lambda i: (i, 0)),
        out_shape=jax.ShapeDtypeStruct((B, D), jnp.float32),
    )(gT, a, b, eye, idx2, tail)


def kernel(mem, idx, val, W_mod):
    V, D = mem.shape
    B = idx.shape[0]
    NW = 32
    n_ch = B // (NW * _CH)
    idx2d = idx.reshape(NW * n_ch, _CH)
    rid16 = jnp.broadcast_to(
        jnp.arange(B, dtype=jnp.int32)[:, None], (B, 16)
    )
    zeros = jnp.zeros((B, D), jnp.float32)
    F0 = 40
    memT = mem.T
    memf = _make_detile(V, D, NW, F0)(memT)
    memfB = _tc_detile(memT, F0)
    w32 = _modulate(val, W_mod)
    s2 = _make_winner_scatter(V, B, NW, n_ch)(idx2d, rid16)
    win16 = _make_win_gather(V, B, NW, n_ch)(s2, idx2d)
    win = win16[:, 0].reshape(NW, n_ch, _CH)
    accA, accB = _make_accumulate(V, B, D, NW, n_ch)(win, w32, zeros)
    gT, ga, gb = _make_gather(V, D, B, NW, n_ch, F0)(
        memf, memfB, idx2d, win, accA, accB)
    n_tail = V - (V // 128) * 128
    tail = mem[V - n_tail:, :]
    return _final_combine(gT, ga, gb, idx.reshape(B, 1), tail, V)


# speedup vs baseline: 1.0078x; 1.0078x over previous
import functools

import jax
import jax.numpy as jnp
from jax import lax
from jax.experimental import pallas as pl
from jax.experimental.pallas import tpu as pltpu
from jax.experimental.pallas import tpu_sc as plsc

_DECAY = 0.99
_CH = 128
_SCP = pltpu.CompilerParams(use_tc_tiling_on_sc=False)


def _wid():
    return lax.axis_index("s") * 2 + lax.axis_index("c")


def _mesh():
    return plsc.VectorSubcoreMesh(core_axis_name="c", subcore_axis_name="s")


def _modulate(val, W):
    B, D = val.shape
    BLK = 2048

    def body(val_ref, w_ref, o_ref):
        o_ref[...] = jnp.tanh(
            jnp.dot(val_ref[...], w_ref[...], preferred_element_type=jnp.float32)
        )

    return pl.pallas_call(
        body,
        grid=(B // BLK,),
        in_specs=[
            pl.BlockSpec((BLK, D), lambda i: (i, 0)),
            pl.BlockSpec((D, D), lambda i: (0, 0)),
        ],
        out_specs=pl.BlockSpec((BLK, D), lambda i: (i, 0)),
        out_shape=jax.ShapeDtypeStruct((B, D), jnp.float32),
    )(val, W)


def _make_winner_scatter(V, B, NW, n_ch):
    @functools.partial(
        pl.kernel,
        mesh=_mesh(),
        compiler_params=_SCP,
        out_type=jax.ShapeDtypeStruct((V, 16), jnp.int32),
        scratch_types=[
            pltpu.VMEM((n_ch, _CH), jnp.int32),
            pltpu.VMEM((n_ch * _CH, 16), jnp.int32),
            pltpu.SemaphoreType.DMA,
        ],
    )
    def k(idx_hbm, rid_hbm, s2_hbm, idx_v, rid_v, sem):
        wid = _wid()
        pltpu.sync_copy(idx_hbm.at[pl.ds(wid * n_ch, n_ch)], idx_v)
        pltpu.sync_copy(rid_hbm.at[pl.ds(wid * n_ch * _CH, n_ch * _CH)], rid_v)
        cps = [
            pltpu.async_copy(
                rid_v.at[pl.ds(c * _CH, _CH)], s2_hbm.at[idx_v.at[c]], sem
            )
            for c in range(n_ch)
        ]
        for cp in cps:
            cp.wait()

    return k


def _make_win_gather(V, B, NW, n_ch):
    b_per_w = n_ch * _CH

    @functools.partial(
        pl.kernel,
        mesh=_mesh(),
        compiler_params=_SCP,
        out_type=jax.ShapeDtypeStruct((B, 16), jnp.int32),
        scratch_types=[
            pltpu.VMEM((n_ch, _CH), jnp.int32),
            pltpu.VMEM((b_per_w, 16), jnp.int32),
            pltpu.SemaphoreType.DMA,
        ],
    )
    def k(s2_hbm, idx_hbm, w16_hbm, idx_v, w16_v, sem):
        wid = _wid()
        pltpu.sync_copy(idx_hbm.at[pl.ds(wid * n_ch, n_ch)], idx_v)
        cps = [
            pltpu.async_copy(
                s2_hbm.at[idx_v.at[c]], w16_v.at[pl.ds(c * _CH, _CH)], sem
            )
            for c in range(n_ch)
        ]
        for cp in cps:
            cp.wait()
        pltpu.sync_copy(w16_v, w16_hbm.at[pl.ds(wid * b_per_w, b_per_w)])

    return k


def _make_accumulate(V, B, D, NW, n_ch):
    b_per_w = n_ch * _CH
    rows_per_tile = B // 16

    @functools.partial(
        pl.kernel,
        mesh=_mesh(),
        compiler_params=_SCP,
        out_type=[
            jax.ShapeDtypeStruct((B, D), jnp.float32),
            jax.ShapeDtypeStruct((B, D), jnp.float32),
        ],
        scratch_types=[
            pltpu.VMEM((n_ch, _CH), jnp.int32),
            pltpu.VMEM((b_per_w, D), jnp.float32),
            pltpu.VMEM_SHARED((B, D), jnp.float32),
            pltpu.SemaphoreType.DMA,
        ],
    )
    def k(win_hbm, w_hbm, z_hbm, accA, accB, win_v, w_v, acc_sp, sem):
        cc = lax.axis_index("c")
        ss = lax.axis_index("s")
        wid = ss * 2 + cc
        pltpu.sync_copy(
            z_hbm.at[pl.ds(ss * rows_per_tile, rows_per_tile)],
            acc_sp.at[pl.ds(ss * rows_per_tile, rows_per_tile)],
        )
        pltpu.sync_copy(win_hbm.at[wid], win_v)
        pltpu.sync_copy(w_hbm.at[pl.ds(wid * b_per_w, b_per_w)], w_v)
        plsc.subcore_barrier()
        for c in range(n_ch):
            pltpu.sync_copy(
                w_v.at[pl.ds(c * _CH, _CH)], acc_sp.at[win_v.at[c]], add=True
            )
        plsc.subcore_barrier()

        @pl.when(cc == 0)
        def _():
            pltpu.sync_copy(
                acc_sp.at[pl.ds(ss * rows_per_tile, rows_per_tile)],
                accA.at[pl.ds(ss * rows_per_tile, rows_per_tile)],
            )

        @pl.when(cc == 1)
        def _():
            pltpu.sync_copy(
                acc_sp.at[pl.ds(ss * rows_per_tile, rows_per_tile)],
                accB.at[pl.ds(ss * rows_per_tile, rows_per_tile)],
            )

    return k


def _tc_detile(memT, F0):
    D, V = memT.shape
    VA = (V // 128) * 128
    NCB = 12
    BLKC = VA // NCB
    FH = D - F0

    def body(memT_ref, out_ref, buf0, buf1, buf2, buf3, s0, s1, s2, s3):
        cb = pl.program_id(0)
        bufs = (buf0, buf1, buf2, buf3)
        sems = (s0, s1, s2, s3)
        rs = []
        for fg in range(FH // 8):
            rs.append(
                pltpu.make_async_copy(
                    memT_ref.at[pl.ds(F0 + fg * 8, 8), pl.ds(cb * BLKC, BLKC)],
                    bufs[fg],
                    sems[fg],
                )
            )
            rs[fg].start()
        ws = []
        for fg in range(FH // 8):
            rs[fg].wait()
            for fi in range(8):
                f = fg * 8 + fi
                w = pltpu.make_async_copy(
                    bufs[fg].at[fi],
                    out_ref.at[pl.ds(f * VA + cb * BLKC, BLKC)],
                    sems[fg],
                )
                w.start()
                ws.append(w)
        for w in ws:
            w.wait()

    return pl.pallas_call(
        body,
        grid=(NCB,),
        in_specs=[pl.BlockSpec(memory_space=pl.ANY)],
        out_specs=pl.BlockSpec(memory_space=pl.ANY),
        out_shape=jax.ShapeDtypeStruct((FH * VA,), jnp.float32),
        scratch_shapes=[
            pltpu.VMEM((8, BLKC), jnp.float32),
            pltpu.VMEM((8, BLKC), jnp.float32),
            pltpu.VMEM((8, BLKC), jnp.float32),
            pltpu.VMEM((8, BLKC), jnp.float32),
            pltpu.SemaphoreType.DMA,
            pltpu.SemaphoreType.DMA,
            pltpu.SemaphoreType.DMA,
            pltpu.SemaphoreType.DMA,
        ],
    )(memT)


def _make_detile(V, D, NW, FH):
    NG = (V // 128) // NW
    EX = (V // 128) % NW
    W_BIG = (NG + 1) * 128
    W_SML = NG * 128
    TAIL = V - (V // 128) * 128

    @functools.partial(
        pl.kernel,
        mesh=_mesh(),
        compiler_params=pltpu.CompilerParams(use_tc_tiling_on_sc=True),
        out_type=jax.ShapeDtypeStruct((FH * V,), jnp.float32),
        scratch_types=[
            pltpu.VMEM((W_BIG,), jnp.float32),
            pltpu.VMEM((W_BIG,), jnp.float32),
            pltpu.VMEM((W_BIG,), jnp.float32),
            pltpu.VMEM((W_BIG,), jnp.float32),
            pltpu.SemaphoreType.DMA,
            pltpu.SemaphoreType.DMA,
            pltpu.SemaphoreType.DMA,
            pltpu.SemaphoreType.DMA,
        ],
    )
    def k(memT_hbm, memf_hbm, buf0, buf1, buf2, buf3, sem0, sem1, sem2, sem3):
        wid = _wid()
        base_big = wid * W_BIG
        base_sml = EX * W_BIG + (wid - EX) * W_SML

        def do(W, base):
            bufs = (buf0, buf1, buf2, buf3)
            sems = (sem0, sem1, sem2, sem3)

            def step(f4, carry):
                rs = []
                for i in range(4):
                    rs.append(pltpu.async_copy(
                        memT_hbm.at[4 * f4 + i].at[pl.ds(base, W)],
                        bufs[i].at[pl.ds(0, W)], sems[i]))
                ws = []
                for i in range(4):
                    rs[i].wait()
                    ws.append(pltpu.async_copy(
                        bufs[i].at[pl.ds(0, W)],
                        memf_hbm.at[pl.ds((4 * f4 + i) * V + base, W)],
                        sems[i]))
                for w in ws:
                    w.wait()
                return carry

            lax.fori_loop(0, FH // 4, step, 0)

        @pl.when(wid < EX)
        def _():
            do(W_BIG, base_big)

        @pl.when(wid >= EX)
        def _():
            do(W_SML, base_sml)

    return k


def _make_gather(V, D, B, NW, n_ch, F0):
    b_per_w = n_ch * _CH
    VA = (V // 128) * 128
    F_UNROLL = 8

    @functools.partial(
        pl.kernel,
        mesh=_mesh(),
        compiler_params=_SCP,
        out_type=[
            jax.ShapeDtypeStruct((D, B), jnp.float32),
            jax.ShapeDtypeStruct((B, D), jnp.float32),
            jax.ShapeDtypeStruct((B, D), jnp.float32),
        ],
        scratch_types=[
            pltpu.VMEM((n_ch, _CH), jnp.int32),
            pltpu.VMEM((n_ch, _CH), jnp.int32),
            pltpu.VMEM((F_UNROLL * n_ch, _CH), jnp.int32),
            pltpu.VMEM((D, b_per_w), jnp.float32),
            pltpu.VMEM((b_per_w, D), jnp.float32),
            pltpu.VMEM((b_per_w, D), jnp.float32),
            pltpu.SemaphoreType.DMA,
            pltpu.SemaphoreType.DMA,
        ],
    )
    def k(memf_hbm, memfB_hbm, idx_hbm, win_hbm, accA, accB,
          gT_hbm, a_hbm, b_hbm,
          idx_v, win_v, idxo_v, g_v, a_v, b_v, sem, sem2):
        wid = _wid()
        pltpu.sync_copy(idx_hbm.at[pl.ds(wid * n_ch, n_ch)], idx_v)
        pltpu.sync_copy(win_hbm.at[wid], win_v)
        acps = []
        for c in range(n_ch):
            acps.append(
                pltpu.async_copy(
                    accA.at[win_v.at[c]], a_v.at[pl.ds(c * _CH, _CH)], sem2
                )
            )
            acps.append(
                pltpu.async_copy(
                    accB.at[win_v.at[c]], b_v.at[pl.ds(c * _CH, _CH)], sem2
                )
            )

        def make_fblock(lo_half):
            def fblock(fb, carry):
                cps = []
                for ff in range(F_UNROLL):
                    for c in range(n_ch):
                        for q in range(_CH // 16):
                            sl = pl.ds(q * 16, 16)
                            if lo_half:
                                f = fb * F_UNROLL + ff
                                idxo_v[ff * n_ch + c, sl] = (
                                    idx_v[c, sl] + f * V
                                )
                            else:
                                f = F0 + fb * F_UNROLL + ff
                                idxo_v[ff * n_ch + c, sl] = (
                                    jnp.minimum(idx_v[c, sl], VA - 1)
                                    + (f - F0) * VA
                                )
                    for c in range(n_ch):
                        src = memf_hbm if lo_half else memfB_hbm
                        fo = (fb * F_UNROLL + ff) if lo_half \
                            else (F0 + fb * F_UNROLL + ff)
                        cps.append(
                            pltpu.async_copy(
                                src.at[idxo_v.at[ff * n_ch + c]],
                                g_v.at[fo].at[pl.ds(c * _CH, _CH)],
                                sem,
                            )
                        )
                for cp in cps:
                    cp.wait()
                return carry

            return fblock

        lax.fori_loop(0, F0 // F_UNROLL, make_fblock(True), 0)
        lax.fori_loop(0, (D - F0) // F_UNROLL, make_fblock(False), 0)
        pltpu.sync_copy(g_v, gT_hbm.at[:, pl.ds(wid * b_per_w, b_per_w)])
        for cp in acps:
            cp.wait()
        pltpu.sync_copy(a_v, a_hbm.at[pl.ds(wid * b_per_w, b_per_w)])
        pltpu.sync_copy(b_v, b_hbm.at[pl.ds(wid * b_per_w, b_per_w)])

    return k


def _final_combine(gT, a, b, idx2, tail, V):
    D, B = gT.shape
    T = tail.shape[0]
    BLK = 2048
    eye = jnp.eye(D, dtype=jnp.float32)

    def body(gT_ref, a_ref, b_ref, eye_ref, idx_ref, tail_ref, o_ref):
        gt = lax.dot_general(
            gT_ref[...], eye_ref[...],
            (((0,), (0,)), ((), ())),
            precision=lax.Precision.HIGHEST,
            preferred_element_type=jnp.float32,
        )
        rel = idx_ref[...] - (V - T)
        oneh = (rel == lax.broadcasted_iota(jnp.int32, (BLK, T), 1)
                ).astype(jnp.float32)
        tg = lax.dot_general(
            oneh, tail_ref[...],
            (((1,), (0,)), ((), ())),
            precision=lax.Precision.HIGHEST,
            preferred_element_type=jnp.float32,
        )
        mask = rel >= 0
        o_ref[...] = _DECAY * jnp.where(mask, tg, gt) + a_ref[...] + b_ref[...]

    return pl.pallas_call(
        body,
        grid=(B // BLK,),
        in_specs=[
            pl.BlockSpec((D, BLK), lambda i: (0, i)),
            pl.BlockSpec((BLK, D), lambda i: (i, 0)),
            pl.BlockSpec((BLK, D), lambda i: (i, 0)),
            pl.BlockSpec((D, D), lambda i: (0, 0)),
            pl.BlockSpec((BLK, 1), lambda i: (i, 0)),
            pl.BlockSpec((T, D), lambda i: (0, 0)),
        ],
        out_specs=pl.BlockSpec((BLK, D), lambda i: (i, 0)),
        out_shape=jax.ShapeDtypeStruct((B, D), jnp.float32),
    )(gT, a, b, eye, idx2, tail)


def kernel(mem, idx, val, W_mod):
    V, D = mem.shape
    B = idx.shape[0]
    NW = 32
    n_ch = B // (NW * _CH)
    idx2d = idx.reshape(NW * n_ch, _CH)
    rid16 = jnp.broadcast_to(
        jnp.arange(B, dtype=jnp.int32)[:, None], (B, 16)
    )
    zeros = jnp.zeros((B, D), jnp.float32)
    F0 = D // 2
    memT = mem.T
    memf = _make_detile(V, D, NW, F0)(memT)
    memfB = _tc_detile(memT, F0)
    w32 = _modulate(val, W_mod)
    s2 = _make_winner_scatter(V, B, NW, n_ch)(idx2d, rid16)
    win16 = _make_win_gather(V, B, NW, n_ch)(s2, idx2d)
    win = win16[:, 0].reshape(NW, n_ch, _CH)
    accA, accB = _make_accumulate(V, B, D, NW, n_ch)(win, w32, zeros)
    gT, ga, gb = _make_gather(V, D, B, NW, n_ch, F0)(
        memf, memfB, idx2d, win, accA, accB)
    n_tail = V - (V // 128) * 128
    tail = mem[V - n_tail:, :]
    return _final_combine(gT, ga, gb, idx.reshape(B, 1), tail, V)
